# Initial kernel scaffold; baseline (speedup 1.0000x reference)
import jax
import jax.numpy as jnp
from jax.experimental import pallas as pl


def kernel(class_logits, pred_bboxes):
    raise NotImplementedError("write your pallas kernel here")


# speedup vs baseline: 2.2847x; 2.2847x over previous
import jax
import jax.numpy as jnp
from jax import lax
from jax.experimental import pallas as pl

B = 4
N = 150
C = 150
P = N * (N - 1)
W = 2 * (4 + C + C + 3)


def _table_body(logits_ref, boxes_ref, rx_ref, ry_ref):
    l = logits_ref[0]
    box = boxes_ref[0]
    s = jax.nn.sigmoid(l)
    soft = jnp.minimum(1.0 - s, s)
    m = jnp.max(s, axis=-1, keepdims=True)
    lp = jnp.log(m + 1e-08)
    ln = jnp.log(1.0 - m + 1e-08)
    z4 = jnp.zeros((N, 4), jnp.float32)
    zC = jnp.zeros((N, C), jnp.float32)
    z1 = jnp.zeros((N, 1), jnp.float32)
    rx_ref[0] = jnp.concatenate(
        [box, z4, s, zC, soft, zC, lp, z1, ln, z1, m, z1], axis=-1)
    ry_ref[0] = jnp.concatenate(
        [z4, box, zC, s, zC, soft, z1, lp, z1, ln, z1, m], axis=-1)


def _expand_body(rx_ref, ry_ref, out_ref):
    i = pl.program_id(1)
    rowx = rx_ref[0, i, :].reshape(1, W)
    ry = ry_ref[0]
    lo = ry[0:N - 1, :]
    hi = ry[1:N, :]
    j = lax.broadcasted_iota(jnp.int32, (N - 1, W), 0)
    out_ref[0, 0] = jnp.where(j < i, lo, hi) + rowx


def kernel(class_logits, pred_bboxes):
    rx, ry = pl.pallas_call(
        _table_body,
        grid=(B,),
        in_specs=[
            pl.BlockSpec((1, N, C), lambda b: (b, 0, 0)),
            pl.BlockSpec((1, N, 4), lambda b: (b, 0, 0)),
        ],
        out_specs=[
            pl.BlockSpec((1, N, W), lambda b: (b, 0, 0)),
            pl.BlockSpec((1, N, W), lambda b: (b, 0, 0)),
        ],
        out_shape=[
            jax.ShapeDtypeStruct((B, N, W), jnp.float32),
            jax.ShapeDtypeStruct((B, N, W), jnp.float32),
        ],
    )(class_logits, pred_bboxes)

    out = pl.pallas_call(
        _expand_body,
        grid=(B, N),
        in_specs=[
            pl.BlockSpec((1, N, W), lambda b, i: (b, 0, 0)),
            pl.BlockSpec((1, N, W), lambda b, i: (b, 0, 0)),
        ],
        out_specs=pl.BlockSpec((1, 1, N - 1, W), lambda b, i: (b, i, 0, 0)),
        out_shape=jax.ShapeDtypeStruct((B, N, N - 1, W), jnp.float32),
    )(rx, ry)
    return out.reshape(B, P, W)


# speedup vs baseline: 9.7988x; 4.2889x over previous
import jax
import jax.numpy as jnp
from jax import lax
from jax.experimental import pallas as pl

B = 4
N = 150
C = 150
P = N * (N - 1)
W = 2 * (4 + C + C + 3)


def _table_body(logits_ref, boxes_ref, rx_ref, ry_ref):
    l = logits_ref[0]
    box = boxes_ref[0]
    s = jax.nn.sigmoid(l)
    soft = jnp.minimum(1.0 - s, s)
    m = jnp.max(s, axis=-1, keepdims=True)
    lp = jnp.log(m + 1e-08)
    ln = jnp.log(1.0 - m + 1e-08)
    z4 = jnp.zeros((N, 4), jnp.float32)
    zC = jnp.zeros((N, C), jnp.float32)
    z1 = jnp.zeros((N, 1), jnp.float32)
    rx_ref[0] = jnp.concatenate(
        [box, z4, s, zC, soft, zC, lp, z1, ln, z1, m, z1], axis=-1)
    ry_ref[0] = jnp.concatenate(
        [z4, box, zC, s, zC, soft, z1, lp, z1, ln, z1, m], axis=-1)


IPB = 8
RPB = IPB * (N - 1)
NG = (P + RPB - 1) // RPB


def _expand_body(rx_ref, ry_ref, out_ref):
    g = pl.program_id(1)
    ry = ry_ref[0]
    lo = ry[0:N - 1, :]
    hi = ry[1:N, :]
    k = lax.broadcasted_iota(jnp.int32, (N - 1, W), 0)
    for di in range(IPB):
        i = jnp.minimum(g * IPB + di, N - 1)
        rowx = rx_ref[0, i, :].reshape(1, W)
        sub = jnp.where(k < i, lo, hi) + rowx
        out_ref[0, pl.ds(di * (N - 1), N - 1), :] = sub


def kernel(class_logits, pred_bboxes):
    rx, ry = pl.pallas_call(
        _table_body,
        grid=(B,),
        in_specs=[
            pl.BlockSpec((1, N, C), lambda b: (b, 0, 0)),
            pl.BlockSpec((1, N, 4), lambda b: (b, 0, 0)),
        ],
        out_specs=[
            pl.BlockSpec((1, N, W), lambda b: (b, 0, 0)),
            pl.BlockSpec((1, N, W), lambda b: (b, 0, 0)),
        ],
        out_shape=[
            jax.ShapeDtypeStruct((B, N, W), jnp.float32),
            jax.ShapeDtypeStruct((B, N, W), jnp.float32),
        ],
    )(class_logits, pred_bboxes)

    out = pl.pallas_call(
        _expand_body,
        grid=(B, NG),
        in_specs=[
            pl.BlockSpec((1, N, W), lambda b, g: (b, 0, 0)),
            pl.BlockSpec((1, N, W), lambda b, g: (b, 0, 0)),
        ],
        out_specs=pl.BlockSpec((1, RPB, W), lambda b, g: (b, g, 0)),
        out_shape=jax.ShapeDtypeStruct((B, P, W), jnp.float32),
    )(rx, ry)
    return out


# speedup vs baseline: 10.0532x; 1.0260x over previous
import jax
import jax.numpy as jnp
from jax import lax
from jax.experimental import pallas as pl

B = 4
N = 150
C = 150
P = N * (N - 1)
W = 2 * (4 + C + C + 3)


def _table_body(logits_ref, boxes_ref, rx_ref, ry_ref):
    l = logits_ref[0]
    box = boxes_ref[0]
    s = jax.nn.sigmoid(l)
    soft = jnp.minimum(1.0 - s, s)
    m = jnp.max(s, axis=-1, keepdims=True)
    lp = jnp.log(m + 1e-08)
    ln = jnp.log(1.0 - m + 1e-08)
    z4 = jnp.zeros((N, 4), jnp.float32)
    zC = jnp.zeros((N, C), jnp.float32)
    z1 = jnp.zeros((N, 1), jnp.float32)
    rx_ref[0] = jnp.concatenate(
        [box, z4, s, zC, soft, zC, lp, z1, ln, z1, m, z1], axis=-1)
    ry_ref[0] = jnp.concatenate(
        [z4, box, zC, s, zC, soft, z1, lp, z1, ln, z1, m], axis=-1)


IPB = 16
RPB = IPB * (N - 1)
NG = (P + RPB - 1) // RPB


def _expand_body(rx_ref, ry_ref, out_ref):
    g = pl.program_id(1)
    ry = ry_ref[0]
    lo = ry[0:N - 1, :]
    hi = ry[1:N, :]
    k = lax.broadcasted_iota(jnp.int32, (N - 1, W), 0)
    for di in range(IPB):
        i = jnp.minimum(g * IPB + di, N - 1)
        rowx = rx_ref[0, i, :].reshape(1, W)
        sub = jnp.where(k < i, lo, hi) + rowx
        out_ref[0, pl.ds(di * (N - 1), N - 1), :] = sub


def kernel(class_logits, pred_bboxes):
    rx, ry = pl.pallas_call(
        _table_body,
        grid=(B,),
        in_specs=[
            pl.BlockSpec((1, N, C), lambda b: (b, 0, 0)),
            pl.BlockSpec((1, N, 4), lambda b: (b, 0, 0)),
        ],
        out_specs=[
            pl.BlockSpec((1, N, W), lambda b: (b, 0, 0)),
            pl.BlockSpec((1, N, W), lambda b: (b, 0, 0)),
        ],
        out_shape=[
            jax.ShapeDtypeStruct((B, N, W), jnp.float32),
            jax.ShapeDtypeStruct((B, N, W), jnp.float32),
        ],
    )(class_logits, pred_bboxes)

    out = pl.pallas_call(
        _expand_body,
        grid=(B, NG),
        in_specs=[
            pl.BlockSpec((1, N, W), lambda b, g: (b, 0, 0)),
            pl.BlockSpec((1, N, W), lambda b, g: (b, 0, 0)),
        ],
        out_specs=pl.BlockSpec((1, RPB, W), lambda b, g: (b, g, 0)),
        out_shape=jax.ShapeDtypeStruct((B, P, W), jnp.float32),
    )(rx, ry)
    return out


# speedup vs baseline: 10.0792x; 1.0026x over previous
import jax
import jax.numpy as jnp
from jax import lax
from jax.experimental import pallas as pl
from jax.experimental.pallas import tpu as pltpu

B = 4
N = 150
C = 150
P = N * (N - 1)
W = 2 * (4 + C + C + 3)

IPB = 8
RPB = IPB * (N - 1)
NG = N // IPB
REM = P - NG * RPB


def _table_body(logits_ref, boxes_ref, rx_ref, ry_ref):
    l = logits_ref[0]
    box = boxes_ref[0]
    s = jax.nn.sigmoid(l)
    soft = jnp.minimum(1.0 - s, s)
    m = jnp.max(s, axis=-1, keepdims=True)
    lp = jnp.log(m + 1e-08)
    ln = jnp.log(1.0 - m + 1e-08)
    z4 = jnp.zeros((N, 4), jnp.float32)
    zC = jnp.zeros((N, C), jnp.float32)
    z1 = jnp.zeros((N, 1), jnp.float32)
    rx_ref[0] = jnp.concatenate(
        [box, z4, s, zC, soft, zC, lp, z1, ln, z1, m, z1], axis=-1)
    ry_ref[0] = jnp.concatenate(
        [z4, box, zC, s, zC, soft, z1, lp, z1, ln, z1, m], axis=-1)


def _expand_body(rx_ref, ry_ref, out_ref, buf_ref, sem_ref):
    g = pl.program_id(0)
    b = pl.program_id(1)

    @pl.when(g >= 1)
    def _wait_prev():
        pltpu.make_async_copy(
            buf_ref.at[b],
            out_ref.at[b, pl.ds((g - 1) * RPB, RPB), :],
            sem_ref.at[b],
        ).wait()

    ry = ry_ref[b]
    lo = ry[0:N - 1, :]
    hi = ry[1:N, :]
    k = lax.broadcasted_iota(jnp.int32, (N - 1, W), 0)
    for di in range(IPB):
        i = g * IPB + di
        rowx = rx_ref[b, i, :].reshape(1, W)
        sub = jnp.where(k < i, lo, hi) + rowx
        buf_ref[b, pl.ds(di * (N - 1), N - 1), :] = sub

    pltpu.make_async_copy(
        buf_ref.at[b],
        out_ref.at[b, pl.ds(g * RPB, RPB), :],
        sem_ref.at[b],
    ).start()

    @pl.when(jnp.logical_and(g == NG - 1, b == B - 1))
    def _drain():
        for bb in range(B):
            pltpu.make_async_copy(
                buf_ref.at[bb],
                out_ref.at[bb, pl.ds((NG - 1) * RPB, RPB), :],
                sem_ref.at[bb],
            ).wait()


def _tail_body(rx_ref, ry_ref, _big_ref, out_ref):
    ry = ry_ref[0]
    lo = ry[0:N - 1, :]
    hi = ry[1:N, :]
    k = lax.broadcasted_iota(jnp.int32, (N - 1, W), 0)
    for di in range(IPB):
        i = min(NG * IPB + di, N - 1)
        rowx = rx_ref[0, i, :].reshape(1, W)
        sub = jnp.where(k < i, lo, hi) + rowx
        out_ref[0, pl.ds(di * (N - 1), N - 1), :] = sub


def kernel(class_logits, pred_bboxes):
    rx, ry = pl.pallas_call(
        _table_body,
        grid=(B,),
        in_specs=[
            pl.BlockSpec((1, N, C), lambda b: (b, 0, 0)),
            pl.BlockSpec((1, N, 4), lambda b: (b, 0, 0)),
        ],
        out_specs=[
            pl.BlockSpec((1, N, W), lambda b: (b, 0, 0)),
            pl.BlockSpec((1, N, W), lambda b: (b, 0, 0)),
        ],
        out_shape=[
            jax.ShapeDtypeStruct((B, N, W), jnp.float32),
            jax.ShapeDtypeStruct((B, N, W), jnp.float32),
        ],
    )(class_logits, pred_bboxes)

    big = pl.pallas_call(
        _expand_body,
        grid=(NG, B),
        in_specs=[
            pl.BlockSpec((B, N, W), lambda g, b: (0, 0, 0)),
            pl.BlockSpec((B, N, W), lambda g, b: (0, 0, 0)),
        ],
        out_specs=pl.BlockSpec(memory_space=pl.ANY),
        out_shape=jax.ShapeDtypeStruct((B, P, W), jnp.float32),
        scratch_shapes=[
            pltpu.VMEM((B, RPB, W), jnp.float32),
            pltpu.SemaphoreType.DMA((B,)),
        ],
    )(rx, ry)

    out = pl.pallas_call(
        _tail_body,
        grid=(B,),
        in_specs=[
            pl.BlockSpec((1, N, W), lambda b: (b, 0, 0)),
            pl.BlockSpec((1, N, W), lambda b: (b, 0, 0)),
            pl.BlockSpec(memory_space=pl.ANY),
        ],
        out_specs=pl.BlockSpec((1, RPB, W), lambda b: (b, NG, 0)),
        out_shape=jax.ShapeDtypeStruct((B, P, W), jnp.float32),
        input_output_aliases={2: 0},
    )(rx, ry, big)
    return out
